# Initial kernel scaffold; baseline (speedup 1.0000x reference)
import jax
import jax.numpy as jnp
from jax.experimental import pallas as pl


def kernel(x, edge_index, train_mask, target_labels, W1, b1, W2, b2, P1, pb1, P2, pb2):
    raise NotImplementedError("write your pallas kernel here")


# speedup vs baseline: 21.2819x; 21.2819x over previous
import functools

import numpy as np

import jax
import jax.numpy as jnp
from jax import lax
from jax.experimental import pallas as pl
from jax.experimental.pallas import tpu as pltpu
from jax.experimental.pallas import tpu_sc as plsc

D = 128
HD = D // 2
LN = 16
NC = 2
NS = 16
CH = 128
RB = 1024


def _sc_degree(dst3, n_pad):
    cpt = dst3.shape[1]
    rpt = n_pad // NS
    mesh = plsc.VectorSubcoreMesh(core_axis_name="c", subcore_axis_name="s")

    @functools.partial(
        pl.kernel,
        out_type=jax.ShapeDtypeStruct((NC, n_pad), jnp.float32),
        mesh=mesh,
        scratch_types=[
            pltpu.VMEM((cpt, CH), jnp.int32),
            pltpu.VMEM((CH,), jnp.float32),
            pltpu.VMEM((rpt,), jnp.float32),
            pltpu.VMEM_SHARED((n_pad,), jnp.float32),
        ],
        compiler_params=pltpu.CompilerParams(use_tc_tiling_on_sc=False),
    )
    def deg_kernel(dst_hbm, out_hbm, dst_v, ones_v, zbuf, acc):
        cid = lax.axis_index("c")
        sid = lax.axis_index("s")
        pltpu.sync_copy(dst_hbm.at[sid], dst_v)

        def zb(i, c):
            zbuf[pl.ds(i * LN, LN)] = jnp.zeros((LN,), jnp.float32)
            return c

        lax.fori_loop(0, rpt // LN, zb, 0)

        def ob(i, c):
            ones_v[pl.ds(i * LN, LN)] = jnp.ones((LN,), jnp.float32)
            return c

        lax.fori_loop(0, CH // LN, ob, 0)
        pltpu.sync_copy(zbuf, acc.at[pl.ds(sid * rpt, rpt)])
        plsc.subcore_barrier()

        def body(i, k):
            c = 2 * i + cid
            pltpu.sync_copy(ones_v, acc.at[dst_v.at[c]], add=True)
            return k

        lax.fori_loop(0, cpt // 2, body, 0)
        plsc.subcore_barrier()
        pltpu.sync_copy(acc.at[pl.ds(sid * rpt, rpt)],
                        out_hbm.at[cid, pl.ds(sid * rpt, rpt)])

    return deg_kernel(dst3)


def _sc_scatter(xs, src3, dst3, zeros, n_pad):
    cpt = src3.shape[1]
    rpt = n_pad // NS
    mesh = plsc.VectorSubcoreMesh(core_axis_name="c", subcore_axis_name="s")

    @functools.partial(
        pl.kernel,
        out_type=jax.ShapeDtypeStruct((NC, n_pad, HD), jnp.float32),
        mesh=mesh,
        scratch_types=[
            pltpu.VMEM((cpt, CH), jnp.int32),
            pltpu.VMEM((cpt, CH), jnp.int32),
            pltpu.VMEM((CH, HD), jnp.float32),
            pltpu.VMEM((CH, HD), jnp.float32),
            pltpu.VMEM_SHARED((n_pad, HD), jnp.float32),
            pltpu.SemaphoreType.DMA,
            pltpu.SemaphoreType.DMA,
        ],
        compiler_params=pltpu.CompilerParams(use_tc_tiling_on_sc=False),
    )
    def scat_kernel(xs_hbm, src_hbm, dst_hbm, zero_hbm, out_hbm,
                    src_v, dst_v, buf0, buf1, acc, sem0, sem1):
        cid = lax.axis_index("c")
        sid = lax.axis_index("s")
        r0 = sid * rpt
        tab = xs_hbm.at[cid]
        pltpu.sync_copy(src_hbm.at[sid], src_v)
        pltpu.sync_copy(dst_hbm.at[sid], dst_v)
        pltpu.sync_copy(zero_hbm.at[pl.ds(r0, rpt)], acc.at[pl.ds(r0, rpt)])
        plsc.subcore_barrier()

        pltpu.make_async_copy(tab.at[src_v.at[0]], buf0, sem0).start()

        def body(j, k):
            c0 = 2 * j
            c1 = 2 * j + 1
            pltpu.make_async_copy(tab.at[src_v.at[c0]], buf0, sem0).wait()
            pltpu.make_async_copy(tab.at[src_v.at[c1]], buf1, sem1).start()
            pltpu.sync_copy(buf0, acc.at[dst_v.at[c0]], add=True)
            pltpu.make_async_copy(tab.at[src_v.at[c1]], buf1, sem1).wait()
            c2 = jnp.minimum(c1 + 1, cpt - 1)
            pltpu.make_async_copy(tab.at[src_v.at[c2]], buf0, sem0).start()
            pltpu.sync_copy(buf1, acc.at[dst_v.at[c1]], add=True)
            return k

        lax.fori_loop(0, cpt // 2, body, 0)
        pltpu.make_async_copy(tab.at[src_v.at[0]], buf0, sem0).wait()
        plsc.subcore_barrier()
        pltpu.sync_copy(acc.at[pl.ds(r0, rpt)],
                        out_hbm.at[cid, pl.ds(r0, rpt)])

    return scat_kernel(xs, src3, dst3, zeros)


def _dinv_of(deg_ref):
    deg = deg_ref[0, :] + deg_ref[1, :] + 1.0
    return lax.rsqrt(deg)[:, None]


def _split_store(o_ref, v):
    o_ref[0, :, :] = v[:, :HD]
    o_ref[1, :, :] = v[:, HD:]


def _tc_embed1(x_pad, W1, deg_part, n_pad):
    def body(x_ref, w_ref, deg_ref, o_ref):
        xw = jnp.dot(x_ref[...], w_ref[...], preferred_element_type=jnp.float32)
        _split_store(o_ref, xw * _dinv_of(deg_ref))

    return pl.pallas_call(
        body,
        grid=(n_pad // RB,),
        in_specs=[
            pl.BlockSpec((RB, D), lambda i: (i, 0)),
            pl.BlockSpec((D, D), lambda i: (0, 0)),
            pl.BlockSpec((NC, RB), lambda i: (0, i)),
        ],
        out_specs=pl.BlockSpec((NC, RB, HD), lambda i: (0, i, 0)),
        out_shape=jax.ShapeDtypeStruct((NC, n_pad, HD), jnp.float32),
    )(x_pad, W1, deg_part)


def _tc_layer2(part1, xs1, deg_part, W2, b1, n_pad):
    def body(p_ref, xs_ref, deg_ref, w_ref, b_ref, o_ref):
        dinv = _dinv_of(deg_ref)
        h0 = jnp.maximum(
            dinv * (p_ref[0] + xs_ref[0]) + b_ref[0, :HD], 0.0)
        h1 = jnp.maximum(
            dinv * (p_ref[1] + xs_ref[1]) + b_ref[0, HD:], 0.0)
        hw = (jnp.dot(h0, w_ref[:HD, :], preferred_element_type=jnp.float32)
              + jnp.dot(h1, w_ref[HD:, :], preferred_element_type=jnp.float32))
        _split_store(o_ref, hw * dinv)

    return pl.pallas_call(
        body,
        grid=(n_pad // RB,),
        in_specs=[
            pl.BlockSpec((NC, RB, HD), lambda i: (0, i, 0)),
            pl.BlockSpec((NC, RB, HD), lambda i: (0, i, 0)),
            pl.BlockSpec((NC, RB), lambda i: (0, i)),
            pl.BlockSpec((D, D), lambda i: (0, 0)),
            pl.BlockSpec((1, D), lambda i: (0, 0)),
        ],
        out_specs=pl.BlockSpec((NC, RB, HD), lambda i: (0, i, 0)),
        out_shape=jax.ShapeDtypeStruct((NC, n_pad, HD), jnp.float32),
    )(part1, xs1, deg_part, W2, b1)


def _tc_head(part2, xs2, deg_part, b2, P1, pb1, P2, pb2, y_pad, m_pad, n_pad):
    od = P2.shape[1]

    def body(p_ref, xs_ref, deg_ref, b2_ref, P1_ref, pb1_ref,
             P2_ref, pb2_ref, y_ref, m_ref, lo_ref, s_ref, c_ref):
        dinv = _dinv_of(deg_ref)
        ne0 = dinv * (p_ref[0] + xs_ref[0]) + b2_ref[0, :HD]
        ne1 = dinv * (p_ref[1] + xs_ref[1]) + b2_ref[0, HD:]
        z = jnp.maximum(
            jnp.dot(ne0, P1_ref[:HD, :], preferred_element_type=jnp.float32)
            + jnp.dot(ne1, P1_ref[HD:, :], preferred_element_type=jnp.float32)
            + pb1_ref[...], 0.0)
        logits = jnp.dot(
            z, P2_ref[...], preferred_element_type=jnp.float32) + pb2_ref[...]
        lo_ref[...] = logits
        m = m_ref[...]
        bce = (jnp.maximum(logits, 0.0) - logits * y_ref[...]
               + jnp.log1p(jnp.exp(-jnp.abs(logits))))

        @pl.when(pl.program_id(0) == 0)
        def _():
            s_ref[...] = jnp.zeros_like(s_ref)
            c_ref[...] = jnp.zeros_like(c_ref)

        s_ref[...] += jnp.sum(bce * m)
        c_ref[...] += jnp.sum(m)

    return pl.pallas_call(
        body,
        grid=(n_pad // RB,),
        in_specs=[
            pl.BlockSpec((NC, RB, HD), lambda i: (0, i, 0)),
            pl.BlockSpec((NC, RB, HD), lambda i: (0, i, 0)),
            pl.BlockSpec((NC, RB), lambda i: (0, i)),
            pl.BlockSpec((1, D), lambda i: (0, 0)),
            pl.BlockSpec((D, D), lambda i: (0, 0)),
            pl.BlockSpec((1, D), lambda i: (0, 0)),
            pl.BlockSpec((D, od), lambda i: (0, 0)),
            pl.BlockSpec((1, od), lambda i: (0, 0)),
            pl.BlockSpec((RB, od), lambda i: (i, 0)),
            pl.BlockSpec((RB, 1), lambda i: (i, 0)),
        ],
        out_specs=[
            pl.BlockSpec((RB, od), lambda i: (i, 0)),
            pl.BlockSpec((1, 1), lambda i: (0, 0)),
            pl.BlockSpec((1, 1), lambda i: (0, 0)),
        ],
        out_shape=[
            jax.ShapeDtypeStruct((n_pad, od), jnp.float32),
            jax.ShapeDtypeStruct((1, 1), jnp.float32),
            jax.ShapeDtypeStruct((1, 1), jnp.float32),
        ],
    )(part2, xs2, deg_part, b2, P1, pb1, P2, pb2, y_pad, m_pad)


def kernel(x, edge_index, train_mask, target_labels,
           W1, b1, W2, b2, P1, pb1, P2, pb2):
    n = x.shape[0]
    n_pad = -(-n // RB) * RB
    if n_pad == n:
        n_pad += RB
    e = edge_index.shape[1]
    cpt = -(-e // (NS * CH))
    cpt += cpt % 2
    e_pad = NS * CH * cpt
    pad = e_pad - e

    pad_src = np.arange(pad, dtype=np.int32) % n
    pad_dst = n + np.arange(pad, dtype=np.int32) % (n_pad - n)
    src3 = jnp.concatenate(
        [edge_index[0], jnp.asarray(pad_src)]).reshape(NS, cpt, CH)
    dst3 = jnp.concatenate(
        [edge_index[1], jnp.asarray(pad_dst)]).reshape(NS, cpt, CH)

    x_pad = jnp.pad(x, ((0, n_pad - n), (0, 0)))
    zeros = jnp.zeros((n_pad, HD), jnp.float32)

    deg_part = _sc_degree(dst3, n_pad)
    xs1 = _tc_embed1(x_pad, W1, deg_part, n_pad)
    part1 = _sc_scatter(xs1, src3, dst3, zeros, n_pad)
    xs2 = _tc_layer2(part1, xs1, deg_part, W2, b1.reshape(1, D), n_pad)
    part2 = _sc_scatter(xs2, src3, dst3, zeros, n_pad)

    y_pad = jnp.pad(target_labels, ((0, n_pad - n), (0, 0)))
    m_pad = jnp.pad(train_mask.astype(jnp.float32),
                    (0, n_pad - n)).reshape(n_pad, 1)
    logits_pad, s, c = _tc_head(part2, xs2, deg_part,
                                b2.reshape(1, D), P1, pb1.reshape(1, -1),
                                P2, pb2.reshape(1, -1), y_pad, m_pad, n_pad)
    logits = logits_pad[:n]
    loss = s[0, 0] / (c[0, 0] * logits.shape[1])
    return (logits, loss)


# speedup vs baseline: 27.8322x; 1.3078x over previous
import functools

import numpy as np

import jax
import jax.numpy as jnp
from jax import lax
from jax.experimental import pallas as pl
from jax.experimental.pallas import tpu as pltpu
from jax.experimental.pallas import tpu_sc as plsc

D = 128
HD = D // 2
LN = 16
NC = 2
NS = 16
CH = 128
RB = 1024


def _sc_degree(dst3, n_pad):
    cpt = dst3.shape[1]
    rpt = n_pad // NS
    mesh = plsc.VectorSubcoreMesh(core_axis_name="c", subcore_axis_name="s")

    @functools.partial(
        pl.kernel,
        out_type=jax.ShapeDtypeStruct((NC, n_pad), jnp.float32),
        mesh=mesh,
        scratch_types=[
            pltpu.VMEM((cpt, CH), jnp.int32),
            pltpu.VMEM((CH,), jnp.float32),
            pltpu.VMEM((rpt,), jnp.float32),
            pltpu.VMEM_SHARED((n_pad,), jnp.float32),
        ],
        compiler_params=pltpu.CompilerParams(use_tc_tiling_on_sc=False),
    )
    def deg_kernel(dst_hbm, out_hbm, dst_v, ones_v, zbuf, acc):
        cid = lax.axis_index("c")
        sid = lax.axis_index("s")
        pltpu.sync_copy(dst_hbm.at[sid], dst_v)

        def zb(i, c):
            zbuf[pl.ds(i * LN, LN)] = jnp.zeros((LN,), jnp.float32)
            return c

        lax.fori_loop(0, rpt // LN, zb, 0)

        def ob(i, c):
            ones_v[pl.ds(i * LN, LN)] = jnp.ones((LN,), jnp.float32)
            return c

        lax.fori_loop(0, CH // LN, ob, 0)
        pltpu.sync_copy(zbuf, acc.at[pl.ds(sid * rpt, rpt)])
        plsc.subcore_barrier()

        def body(i, k):
            c = 2 * i + cid
            pltpu.sync_copy(ones_v, acc.at[dst_v.at[c]], add=True)
            return k

        lax.fori_loop(0, cpt // 2, body, 0)
        plsc.subcore_barrier()
        pltpu.sync_copy(acc.at[pl.ds(sid * rpt, rpt)],
                        out_hbm.at[cid, pl.ds(sid * rpt, rpt)])

    return deg_kernel(dst3)


def _sc_scatter(xs, src3, dst3, zeros, n_pad):
    cpt = src3.shape[1]
    rpt = n_pad // NS
    mesh = plsc.VectorSubcoreMesh(core_axis_name="c", subcore_axis_name="s")

    nb = 4

    @functools.partial(
        pl.kernel,
        out_type=jax.ShapeDtypeStruct((NC, n_pad, HD), jnp.float32),
        mesh=mesh,
        scratch_types=[
            pltpu.VMEM((cpt, CH), jnp.int32),
            pltpu.VMEM((cpt, CH), jnp.int32),
            [pltpu.VMEM((CH, HD), jnp.float32)] * nb,
            pltpu.VMEM_SHARED((n_pad, HD), jnp.float32),
            [pltpu.SemaphoreType.DMA] * nb,
            [pltpu.SemaphoreType.DMA] * nb,
        ],
        compiler_params=pltpu.CompilerParams(use_tc_tiling_on_sc=False),
    )
    def scat_kernel(xs_hbm, src_hbm, dst_hbm, zero_hbm, out_hbm,
                    src_v, dst_v, bufs, acc, gsem, ssem):
        cid = lax.axis_index("c")
        sid = lax.axis_index("s")
        r0 = sid * rpt
        tab = xs_hbm.at[cid]
        pltpu.sync_copy(src_hbm.at[sid], src_v)
        pltpu.sync_copy(dst_hbm.at[sid], dst_v)
        pltpu.sync_copy(zero_hbm.at[pl.ds(r0, rpt)], acc.at[pl.ds(r0, rpt)])
        plsc.subcore_barrier()

        def gather(c, b):
            return pltpu.make_async_copy(tab.at[src_v.at[c]], bufs[b], gsem[b])

        def scat_start(c, b):
            pltpu.async_copy(bufs[b], acc.at[dst_v.at[c]], ssem[b], add=True)

        def scat_wait(c, b):
            pltpu.make_async_copy(bufs[b], acc.at[dst_v.at[c]], ssem[b]).wait()

        for b in range(nb):
            gather(b, b).start()

        def body(j, k):
            c = nb * j
            for b in range(nb):
                gather(c + b, b).wait()
                scat_start(c + b, b)
            for b in range(nb):
                scat_wait(c + b, b)
                gather(jnp.minimum(c + nb + b, cpt - 1), b).start()
            return k

        lax.fori_loop(0, cpt // nb, body, 0)
        for b in range(nb):
            gather(0, b).wait()
        plsc.subcore_barrier()
        pltpu.sync_copy(acc.at[pl.ds(r0, rpt)],
                        out_hbm.at[cid, pl.ds(r0, rpt)])

    return scat_kernel(xs, src3, dst3, zeros)


def _dinv_of(deg_ref):
    deg = deg_ref[0, :] + deg_ref[1, :] + 1.0
    return lax.rsqrt(deg)[:, None]


def _split_store(o_ref, v):
    o_ref[0, :, :] = v[:, :HD]
    o_ref[1, :, :] = v[:, HD:]


def _tc_embed1(x_pad, W1, deg_part, n_pad):
    def body(x_ref, w_ref, deg_ref, o_ref):
        xw = jnp.dot(x_ref[...], w_ref[...], preferred_element_type=jnp.float32)
        _split_store(o_ref, xw * _dinv_of(deg_ref))

    return pl.pallas_call(
        body,
        grid=(n_pad // RB,),
        in_specs=[
            pl.BlockSpec((RB, D), lambda i: (i, 0)),
            pl.BlockSpec((D, D), lambda i: (0, 0)),
            pl.BlockSpec((NC, RB), lambda i: (0, i)),
        ],
        out_specs=pl.BlockSpec((NC, RB, HD), lambda i: (0, i, 0)),
        out_shape=jax.ShapeDtypeStruct((NC, n_pad, HD), jnp.float32),
    )(x_pad, W1, deg_part)


def _tc_layer2(part1, xs1, deg_part, W2, b1, n_pad):
    def body(p_ref, xs_ref, deg_ref, w_ref, b_ref, o_ref):
        dinv = _dinv_of(deg_ref)
        h0 = jnp.maximum(
            dinv * (p_ref[0] + xs_ref[0]) + b_ref[0, :HD], 0.0)
        h1 = jnp.maximum(
            dinv * (p_ref[1] + xs_ref[1]) + b_ref[0, HD:], 0.0)
        hw = (jnp.dot(h0, w_ref[:HD, :], preferred_element_type=jnp.float32)
              + jnp.dot(h1, w_ref[HD:, :], preferred_element_type=jnp.float32))
        _split_store(o_ref, hw * dinv)

    return pl.pallas_call(
        body,
        grid=(n_pad // RB,),
        in_specs=[
            pl.BlockSpec((NC, RB, HD), lambda i: (0, i, 0)),
            pl.BlockSpec((NC, RB, HD), lambda i: (0, i, 0)),
            pl.BlockSpec((NC, RB), lambda i: (0, i)),
            pl.BlockSpec((D, D), lambda i: (0, 0)),
            pl.BlockSpec((1, D), lambda i: (0, 0)),
        ],
        out_specs=pl.BlockSpec((NC, RB, HD), lambda i: (0, i, 0)),
        out_shape=jax.ShapeDtypeStruct((NC, n_pad, HD), jnp.float32),
    )(part1, xs1, deg_part, W2, b1)


def _tc_head(part2, xs2, deg_part, b2, P1, pb1, P2, pb2, y_pad, m_pad, n_pad):
    od = P2.shape[1]

    def body(p_ref, xs_ref, deg_ref, b2_ref, P1_ref, pb1_ref,
             P2_ref, pb2_ref, y_ref, m_ref, lo_ref, s_ref, c_ref):
        dinv = _dinv_of(deg_ref)
        ne0 = dinv * (p_ref[0] + xs_ref[0]) + b2_ref[0, :HD]
        ne1 = dinv * (p_ref[1] + xs_ref[1]) + b2_ref[0, HD:]
        z = jnp.maximum(
            jnp.dot(ne0, P1_ref[:HD, :], preferred_element_type=jnp.float32)
            + jnp.dot(ne1, P1_ref[HD:, :], preferred_element_type=jnp.float32)
            + pb1_ref[...], 0.0)
        logits = jnp.dot(
            z, P2_ref[...], preferred_element_type=jnp.float32) + pb2_ref[...]
        lo_ref[...] = logits
        m = m_ref[...]
        bce = (jnp.maximum(logits, 0.0) - logits * y_ref[...]
               + jnp.log1p(jnp.exp(-jnp.abs(logits))))

        @pl.when(pl.program_id(0) == 0)
        def _():
            s_ref[...] = jnp.zeros_like(s_ref)
            c_ref[...] = jnp.zeros_like(c_ref)

        s_ref[...] += jnp.sum(bce * m)
        c_ref[...] += jnp.sum(m)

    return pl.pallas_call(
        body,
        grid=(n_pad // RB,),
        in_specs=[
            pl.BlockSpec((NC, RB, HD), lambda i: (0, i, 0)),
            pl.BlockSpec((NC, RB, HD), lambda i: (0, i, 0)),
            pl.BlockSpec((NC, RB), lambda i: (0, i)),
            pl.BlockSpec((1, D), lambda i: (0, 0)),
            pl.BlockSpec((D, D), lambda i: (0, 0)),
            pl.BlockSpec((1, D), lambda i: (0, 0)),
            pl.BlockSpec((D, od), lambda i: (0, 0)),
            pl.BlockSpec((1, od), lambda i: (0, 0)),
            pl.BlockSpec((RB, od), lambda i: (i, 0)),
            pl.BlockSpec((RB, 1), lambda i: (i, 0)),
        ],
        out_specs=[
            pl.BlockSpec((RB, od), lambda i: (i, 0)),
            pl.BlockSpec((1, 1), lambda i: (0, 0)),
            pl.BlockSpec((1, 1), lambda i: (0, 0)),
        ],
        out_shape=[
            jax.ShapeDtypeStruct((n_pad, od), jnp.float32),
            jax.ShapeDtypeStruct((1, 1), jnp.float32),
            jax.ShapeDtypeStruct((1, 1), jnp.float32),
        ],
    )(part2, xs2, deg_part, b2, P1, pb1, P2, pb2, y_pad, m_pad)


def kernel(x, edge_index, train_mask, target_labels,
           W1, b1, W2, b2, P1, pb1, P2, pb2):
    n = x.shape[0]
    n_pad = -(-n // RB) * RB
    if n_pad == n:
        n_pad += RB
    e = edge_index.shape[1]
    cpt = -(-e // (NS * CH))
    cpt = -(-cpt // 4) * 4
    e_pad = NS * CH * cpt
    pad = e_pad - e

    pad_src = np.arange(pad, dtype=np.int32) % n
    pad_dst = n + np.arange(pad, dtype=np.int32) % (n_pad - n)
    src3 = jnp.concatenate(
        [edge_index[0], jnp.asarray(pad_src)]).reshape(NS, cpt, CH)
    dst3 = jnp.concatenate(
        [edge_index[1], jnp.asarray(pad_dst)]).reshape(NS, cpt, CH)

    x_pad = jnp.pad(x, ((0, n_pad - n), (0, 0)))
    zeros = jnp.zeros((n_pad, HD), jnp.float32)

    deg_part = _sc_degree(dst3, n_pad)
    xs1 = _tc_embed1(x_pad, W1, deg_part, n_pad)
    part1 = _sc_scatter(xs1, src3, dst3, zeros, n_pad)
    xs2 = _tc_layer2(part1, xs1, deg_part, W2, b1.reshape(1, D), n_pad)
    part2 = _sc_scatter(xs2, src3, dst3, zeros, n_pad)

    y_pad = jnp.pad(target_labels, ((0, n_pad - n), (0, 0)))
    m_pad = jnp.pad(train_mask.astype(jnp.float32),
                    (0, n_pad - n)).reshape(n_pad, 1)
    logits_pad, s, c = _tc_head(part2, xs2, deg_part,
                                b2.reshape(1, D), P1, pb1.reshape(1, -1),
                                P2, pb2.reshape(1, -1), y_pad, m_pad, n_pad)
    logits = logits_pad[:n]
    loss = s[0, 0] / (c[0, 0] * logits.shape[1])
    return (logits, loss)


# speedup vs baseline: 28.1629x; 1.0119x over previous
import functools

import numpy as np

import jax
import jax.numpy as jnp
from jax import lax
from jax.experimental import pallas as pl
from jax.experimental.pallas import tpu as pltpu
from jax.experimental.pallas import tpu_sc as plsc

D = 128
HD = D // 2
LN = 16
NC = 2
NS = 16
CH = 128
RB = 1024


def _sc_degree(dst3, n_pad):
    cpt = dst3.shape[1]
    rpt = n_pad // NS
    mesh = plsc.VectorSubcoreMesh(core_axis_name="c", subcore_axis_name="s")

    @functools.partial(
        pl.kernel,
        out_type=jax.ShapeDtypeStruct((NC, n_pad), jnp.float32),
        mesh=mesh,
        scratch_types=[
            pltpu.VMEM((cpt, CH), jnp.int32),
            pltpu.VMEM((CH,), jnp.float32),
            pltpu.VMEM((rpt,), jnp.float32),
            pltpu.VMEM_SHARED((n_pad,), jnp.float32),
        ],
        compiler_params=pltpu.CompilerParams(use_tc_tiling_on_sc=False),
    )
    def deg_kernel(dst_hbm, out_hbm, dst_v, ones_v, zbuf, acc):
        cid = lax.axis_index("c")
        sid = lax.axis_index("s")
        pltpu.sync_copy(dst_hbm.at[sid], dst_v)

        def zb(i, c):
            zbuf[pl.ds(i * LN, LN)] = jnp.zeros((LN,), jnp.float32)
            return c

        lax.fori_loop(0, rpt // LN, zb, 0)

        def ob(i, c):
            ones_v[pl.ds(i * LN, LN)] = jnp.ones((LN,), jnp.float32)
            return c

        lax.fori_loop(0, CH // LN, ob, 0)
        pltpu.sync_copy(zbuf, acc.at[pl.ds(sid * rpt, rpt)])
        plsc.subcore_barrier()

        def body(i, k):
            c = 2 * i + cid
            pltpu.sync_copy(ones_v, acc.at[dst_v.at[c]], add=True)
            return k

        lax.fori_loop(0, cpt // 2, body, 0)
        plsc.subcore_barrier()
        pltpu.sync_copy(acc.at[pl.ds(sid * rpt, rpt)],
                        out_hbm.at[cid, pl.ds(sid * rpt, rpt)])

    return deg_kernel(dst3)


def _sc_scatter(xs, src3, dst3, zeros, n_pad):
    cpt = src3.shape[1]
    rpt = n_pad // NS
    mesh = plsc.VectorSubcoreMesh(core_axis_name="c", subcore_axis_name="s")

    nb = 5

    @functools.partial(
        pl.kernel,
        out_type=jax.ShapeDtypeStruct((NC, n_pad, HD), jnp.float32),
        mesh=mesh,
        scratch_types=[
            pltpu.VMEM((cpt, CH), jnp.int32),
            pltpu.VMEM((cpt, CH), jnp.int32),
            [pltpu.VMEM((CH, HD), jnp.float32)] * nb,
            pltpu.VMEM_SHARED((n_pad, HD), jnp.float32),
            [pltpu.SemaphoreType.DMA] * nb,
            [pltpu.SemaphoreType.DMA] * nb,
        ],
        compiler_params=pltpu.CompilerParams(use_tc_tiling_on_sc=False),
    )
    def scat_kernel(xs_hbm, src_hbm, dst_hbm, zero_hbm, out_hbm,
                    src_v, dst_v, bufs, acc, gsem, ssem):
        cid = lax.axis_index("c")
        sid = lax.axis_index("s")
        r0 = sid * rpt
        tab = xs_hbm.at[cid]
        pltpu.sync_copy(src_hbm.at[sid], src_v)
        pltpu.sync_copy(dst_hbm.at[sid], dst_v)
        pltpu.sync_copy(zero_hbm.at[pl.ds(r0, rpt)], acc.at[pl.ds(r0, rpt)])
        plsc.subcore_barrier()

        def gather(c, b):
            return pltpu.make_async_copy(tab.at[src_v.at[c]], bufs[b], gsem[b])

        def scat_start(c, b):
            pltpu.async_copy(bufs[b], acc.at[dst_v.at[c]], ssem[b], add=True)

        def scat_wait(c, b):
            pltpu.make_async_copy(bufs[b], acc.at[dst_v.at[c]], ssem[b]).wait()

        for b in range(nb):
            gather(b, b).start()

        def body(j, k):
            c = nb * j
            for b in range(nb):
                gather(c + b, b).wait()
                scat_start(c + b, b)
            for b in range(nb):
                scat_wait(c + b, b)
                gather(jnp.minimum(c + nb + b, cpt - 1), b).start()
            return k

        lax.fori_loop(0, cpt // nb, body, 0)
        for b in range(nb):
            gather(0, b).wait()
        plsc.subcore_barrier()
        pltpu.sync_copy(acc.at[pl.ds(r0, rpt)],
                        out_hbm.at[cid, pl.ds(r0, rpt)])

    return scat_kernel(xs, src3, dst3, zeros)


def _dinv_of(deg_ref):
    deg = deg_ref[0, :] + deg_ref[1, :] + 1.0
    return lax.rsqrt(deg)[:, None]


def _split_store(o_ref, v):
    o_ref[0, :, :] = v[:, :HD]
    o_ref[1, :, :] = v[:, HD:]


def _tc_embed1(x_pad, W1, deg_part, n_pad):
    def body(x_ref, w_ref, deg_ref, o_ref):
        xw = jnp.dot(x_ref[...], w_ref[...], preferred_element_type=jnp.float32)
        _split_store(o_ref, xw * _dinv_of(deg_ref))

    return pl.pallas_call(
        body,
        grid=(n_pad // RB,),
        in_specs=[
            pl.BlockSpec((RB, D), lambda i: (i, 0)),
            pl.BlockSpec((D, D), lambda i: (0, 0)),
            pl.BlockSpec((NC, RB), lambda i: (0, i)),
        ],
        out_specs=pl.BlockSpec((NC, RB, HD), lambda i: (0, i, 0)),
        out_shape=jax.ShapeDtypeStruct((NC, n_pad, HD), jnp.float32),
    )(x_pad, W1, deg_part)


def _tc_layer2(part1, xs1, deg_part, W2, b1, n_pad):
    def body(p_ref, xs_ref, deg_ref, w_ref, b_ref, o_ref):
        dinv = _dinv_of(deg_ref)
        h0 = jnp.maximum(
            dinv * (p_ref[0] + xs_ref[0]) + b_ref[0, :HD], 0.0)
        h1 = jnp.maximum(
            dinv * (p_ref[1] + xs_ref[1]) + b_ref[0, HD:], 0.0)
        hw = (jnp.dot(h0, w_ref[:HD, :], preferred_element_type=jnp.float32)
              + jnp.dot(h1, w_ref[HD:, :], preferred_element_type=jnp.float32))
        _split_store(o_ref, hw * dinv)

    return pl.pallas_call(
        body,
        grid=(n_pad // RB,),
        in_specs=[
            pl.BlockSpec((NC, RB, HD), lambda i: (0, i, 0)),
            pl.BlockSpec((NC, RB, HD), lambda i: (0, i, 0)),
            pl.BlockSpec((NC, RB), lambda i: (0, i)),
            pl.BlockSpec((D, D), lambda i: (0, 0)),
            pl.BlockSpec((1, D), lambda i: (0, 0)),
        ],
        out_specs=pl.BlockSpec((NC, RB, HD), lambda i: (0, i, 0)),
        out_shape=jax.ShapeDtypeStruct((NC, n_pad, HD), jnp.float32),
    )(part1, xs1, deg_part, W2, b1)


def _tc_head(part2, xs2, deg_part, b2, P1, pb1, P2, pb2, y_pad, m_pad, n_pad):
    od = P2.shape[1]

    def body(p_ref, xs_ref, deg_ref, b2_ref, P1_ref, pb1_ref,
             P2_ref, pb2_ref, y_ref, m_ref, lo_ref, s_ref, c_ref):
        dinv = _dinv_of(deg_ref)
        ne0 = dinv * (p_ref[0] + xs_ref[0]) + b2_ref[0, :HD]
        ne1 = dinv * (p_ref[1] + xs_ref[1]) + b2_ref[0, HD:]
        z = jnp.maximum(
            jnp.dot(ne0, P1_ref[:HD, :], preferred_element_type=jnp.float32)
            + jnp.dot(ne1, P1_ref[HD:, :], preferred_element_type=jnp.float32)
            + pb1_ref[...], 0.0)
        logits = jnp.dot(
            z, P2_ref[...], preferred_element_type=jnp.float32) + pb2_ref[...]
        lo_ref[...] = logits
        m = m_ref[...]
        bce = (jnp.maximum(logits, 0.0) - logits * y_ref[...]
               + jnp.log1p(jnp.exp(-jnp.abs(logits))))

        @pl.when(pl.program_id(0) == 0)
        def _():
            s_ref[...] = jnp.zeros_like(s_ref)
            c_ref[...] = jnp.zeros_like(c_ref)

        s_ref[...] += jnp.sum(bce * m)
        c_ref[...] += jnp.sum(m)

    return pl.pallas_call(
        body,
        grid=(n_pad // RB,),
        in_specs=[
            pl.BlockSpec((NC, RB, HD), lambda i: (0, i, 0)),
            pl.BlockSpec((NC, RB, HD), lambda i: (0, i, 0)),
            pl.BlockSpec((NC, RB), lambda i: (0, i)),
            pl.BlockSpec((1, D), lambda i: (0, 0)),
            pl.BlockSpec((D, D), lambda i: (0, 0)),
            pl.BlockSpec((1, D), lambda i: (0, 0)),
            pl.BlockSpec((D, od), lambda i: (0, 0)),
            pl.BlockSpec((1, od), lambda i: (0, 0)),
            pl.BlockSpec((RB, od), lambda i: (i, 0)),
            pl.BlockSpec((RB, 1), lambda i: (i, 0)),
        ],
        out_specs=[
            pl.BlockSpec((RB, od), lambda i: (i, 0)),
            pl.BlockSpec((1, 1), lambda i: (0, 0)),
            pl.BlockSpec((1, 1), lambda i: (0, 0)),
        ],
        out_shape=[
            jax.ShapeDtypeStruct((n_pad, od), jnp.float32),
            jax.ShapeDtypeStruct((1, 1), jnp.float32),
            jax.ShapeDtypeStruct((1, 1), jnp.float32),
        ],
    )(part2, xs2, deg_part, b2, P1, pb1, P2, pb2, y_pad, m_pad)


def kernel(x, edge_index, train_mask, target_labels,
           W1, b1, W2, b2, P1, pb1, P2, pb2):
    n = x.shape[0]
    n_pad = -(-n // RB) * RB
    if n_pad == n:
        n_pad += RB
    e = edge_index.shape[1]
    cpt = -(-e // (NS * CH))
    cpt = -(-cpt // 40) * 40
    e_pad = NS * CH * cpt
    pad = e_pad - e

    pad_src = np.arange(pad, dtype=np.int32) % n
    pad_dst = n + np.arange(pad, dtype=np.int32) % (n_pad - n)
    src3 = jnp.concatenate(
        [edge_index[0], jnp.asarray(pad_src)]).reshape(NS, cpt, CH)
    dst3 = jnp.concatenate(
        [edge_index[1], jnp.asarray(pad_dst)]).reshape(NS, cpt, CH)

    x_pad = jnp.pad(x, ((0, n_pad - n), (0, 0)))
    zeros = jnp.zeros((n_pad, HD), jnp.float32)

    deg_part = _sc_degree(dst3, n_pad)
    xs1 = _tc_embed1(x_pad, W1, deg_part, n_pad)
    part1 = _sc_scatter(xs1, src3, dst3, zeros, n_pad)
    xs2 = _tc_layer2(part1, xs1, deg_part, W2, b1.reshape(1, D), n_pad)
    part2 = _sc_scatter(xs2, src3, dst3, zeros, n_pad)

    y_pad = jnp.pad(target_labels, ((0, n_pad - n), (0, 0)))
    m_pad = jnp.pad(train_mask.astype(jnp.float32),
                    (0, n_pad - n)).reshape(n_pad, 1)
    logits_pad, s, c = _tc_head(part2, xs2, deg_part,
                                b2.reshape(1, D), P1, pb1.reshape(1, -1),
                                P2, pb2.reshape(1, -1), y_pad, m_pad, n_pad)
    logits = logits_pad[:n]
    loss = s[0, 0] / (c[0, 0] * logits.shape[1])
    return (logits, loss)
